# Initial kernel scaffold; baseline (speedup 1.0000x reference)
import jax
import jax.numpy as jnp
from jax.experimental import pallas as pl


def kernel(x, token_table, pos_table, gamma, beta):
    raise NotImplementedError("write your pallas kernel here")


# speedup vs baseline: 1.4044x; 1.4044x over previous
import functools

import jax
import jax.numpy as jnp
from jax import lax
from jax.experimental import pallas as pl
from jax.experimental.pallas import tpu as pltpu
from jax.experimental.pallas import tpu_sc as plsc

_EPS = 1e-6
_GATHER_WINDOW = 128


def _sc_gather(table, idx2d):
    n_idx = idx2d.shape[1]
    h = table.shape[1]
    mesh = plsc.VectorSubcoreMesh(core_axis_name="c", subcore_axis_name="s")

    @functools.partial(
        pl.kernel,
        out_type=jax.ShapeDtypeStruct((n_idx, h), table.dtype),
        mesh=mesh,
        compiler_params=pltpu.CompilerParams(use_tc_tiling_on_sc=False),
    )
    def gather_kernel(table_hbm, i_hbm, o_hbm):
        def body(i_vmem, o_vmem):
            pltpu.sync_copy(table_hbm.at[i_vmem.at[0]], o_vmem)

        pltpu.emit_pipeline(
            body,
            grid=(n_idx // _GATHER_WINDOW,),
            in_specs=[
                pl.BlockSpec((1, _GATHER_WINDOW), index_map=lambda i: (0, i))
            ],
            out_specs=[
                pl.BlockSpec((_GATHER_WINDOW, h), index_map=lambda i: (i, 0))
            ],
            core_axis_name=("c", "s"),
            dimension_semantics=(pltpu.PARALLEL,),
        )(i_hbm, o_hbm)

    return gather_kernel(table, idx2d)


def _mask_body(x_ref, m_ref):
    bb, l = x_ref.shape
    m = x_ref[...] > 0
    m_ref[...] = jnp.broadcast_to(m[:, None, None, :], (bb, 1, l, l))


def _ln_body(tok_ref, pos_ref, g_ref, b_ref, o_ref):
    h = tok_ref[...] + pos_ref[...][None]
    hidden = h.shape[-1]
    mean = jnp.mean(h, axis=-1, keepdims=True)
    c = h - mean
    var = jnp.sum(c * c, axis=-1, keepdims=True) / (hidden - 1)
    std = jnp.sqrt(var)
    o_ref[...] = g_ref[...][None, None] * (c / (std + _EPS)) + b_ref[...][None, None]


def kernel(x, token_table, pos_table, gamma, beta):
    b, l = x.shape
    _, hidden = token_table.shape

    idx2d = x.reshape(1, b * l).astype(jnp.int32)
    tok = _sc_gather(token_table, idx2d).reshape(b, l, hidden)

    bb = 8
    mask = pl.pallas_call(
        _mask_body,
        grid=(b // bb,),
        in_specs=[pl.BlockSpec((bb, l), lambda i: (i, 0))],
        out_specs=pl.BlockSpec((bb, 1, l, l), lambda i: (i, 0, 0, 0)),
        out_shape=jax.ShapeDtypeStruct((b, 1, l, l), jnp.bool_),
    )(x)

    out = pl.pallas_call(
        _ln_body,
        grid=(b // bb,),
        in_specs=[
            pl.BlockSpec((bb, l, hidden), lambda i: (i, 0, 0)),
            pl.BlockSpec((l, hidden), lambda i: (0, 0)),
            pl.BlockSpec((hidden,), lambda i: (0,)),
            pl.BlockSpec((hidden,), lambda i: (0,)),
        ],
        out_specs=pl.BlockSpec((bb, l, hidden), lambda i: (i, 0, 0)),
        out_shape=jax.ShapeDtypeStruct((b, l, hidden), jnp.float32),
    )(tok, pos_table, gamma, beta)

    return (out, mask)
